# baseline (speedup 1.0000x reference)
import functools

import jax
import jax.numpy as jnp
from jax import lax
from jax.experimental import pallas as pl
from jax.experimental.pallas import tpu as pltpu
from jax.experimental.pallas import tpu_sc as plsc

_M = 1000000
_N = 4
_NW = 32
_ROWS_PER_W = (_M // _NW) // 8 * 8
_TAIL = _M - _NW * _ROWS_PER_W


def _sc_body(t_hbm, w_hbm, ot_hbm, ow_hbm):
    wid = lax.axis_index("s") * 2 + lax.axis_index("c")
    base = wid * _ROWS_PER_W
    sl = pl.ds(base, _ROWS_PER_W)
    pltpu.sync_copy(t_hbm.at[sl], ot_hbm.at[sl])
    pltpu.sync_copy(w_hbm.at[sl], ow_hbm.at[sl])

    @pl.when(wid == 0)
    def _tail():
        tl = pl.ds(_NW * _ROWS_PER_W, _TAIL)
        pltpu.sync_copy(t_hbm.at[tl], ot_hbm.at[tl])
        pltpu.sync_copy(w_hbm.at[tl], ow_hbm.at[tl])


def kernel(bbox_targets, bbox_weights, labels):
    M, N = bbox_targets.shape
    mesh = plsc.VectorSubcoreMesh(core_axis_name="c", subcore_axis_name="s")
    f = functools.partial(
        pl.kernel,
        mesh=mesh,
        out_type=[jax.ShapeDtypeStruct((M, N), jnp.float32)] * 2,
    )(_sc_body)
    return tuple(f(bbox_targets, bbox_weights))


# speedup vs baseline: 20.1482x; 20.1482x over previous
import jax
import jax.numpy as jnp
from jax.experimental import pallas as pl

_BR = 8192


def _copy_body(a_ref, b_ref, oa_ref, ob_ref):
    oa_ref[...] = a_ref[...]
    ob_ref[...] = b_ref[...]


def kernel(bbox_targets, bbox_weights, labels):
    M, N = bbox_targets.shape
    spec = pl.BlockSpec((_BR, N), lambda i: (i, 0))
    out_t, out_w = pl.pallas_call(
        _copy_body,
        grid=(pl.cdiv(M, _BR),),
        in_specs=[spec, spec],
        out_specs=[spec, spec],
        out_shape=[jax.ShapeDtypeStruct((M, N), jnp.float32)] * 2,
    )(bbox_targets, bbox_weights)
    return out_t, out_w


# speedup vs baseline: 20.1504x; 1.0001x over previous
import jax
import jax.numpy as jnp
from jax.experimental import pallas as pl

_BR = 12288


def _copy_body(a_ref, b_ref, oa_ref, ob_ref):
    oa_ref[...] = a_ref[...]
    ob_ref[...] = b_ref[...]


def kernel(bbox_targets, bbox_weights, labels):
    M, N = bbox_targets.shape
    spec = pl.BlockSpec((_BR, N), lambda i: (i, 0))
    out_t, out_w = pl.pallas_call(
        _copy_body,
        grid=(pl.cdiv(M, _BR),),
        in_specs=[spec, spec],
        out_specs=[spec, spec],
        out_shape=[jax.ShapeDtypeStruct((M, N), jnp.float32)] * 2,
    )(bbox_targets, bbox_weights)
    return out_t, out_w
